# Initial kernel scaffold; baseline (speedup 1.0000x reference)
import jax
import jax.numpy as jnp
from jax.experimental import pallas as pl


def kernel(x, W1, b1, W2, b2, codebook, W3, b3, W4, b4):
    raise NotImplementedError("write your pallas kernel here")


# speedup vs baseline: 1.1342x; 1.1342x over previous
import functools

import jax
import jax.numpy as jnp
from jax.experimental import pallas as pl
from jax.experimental.pallas import tpu as pltpu

B = 4096
INPUT_DIM = 1024
HIDDEN_DIM = 2048
K = 8192
D = 32
L = 16

ENC_BT = 512
VQ_BT = 256
DEC_BT = 512


def _encoder_kernel(x_ref, w1_ref, b1_ref, w2_ref, b2_ref, z_ref):
    h = jnp.maximum(
        jnp.dot(x_ref[...], w1_ref[...], preferred_element_type=jnp.float32)
        + b1_ref[...],
        0.0,
    )
    z_ref[...] = (
        jnp.dot(h, w2_ref[...], preferred_element_type=jnp.float32) + b2_ref[...]
    )


def _vq_kernel(z_ref, zsq_ref, csq_ref, cbt_ref, cb_ref, idx_ref, zq_ref):
    iota_k = jax.lax.broadcasted_iota(jnp.int32, (1, K), 1)
    for l in range(L):
        zi = z_ref[:, l * D : (l + 1) * D]
        dot = jnp.dot(
            zi.astype(jnp.bfloat16), cbt_ref[...], preferred_element_type=jnp.float32
        )
        dist = (zsq_ref[:, l : l + 1] + csq_ref[...]) - 2.0 * dot
        minval = jnp.min(dist, axis=1, keepdims=True)
        idx = jnp.min(jnp.where(dist == minval, iota_k, K), axis=1).astype(jnp.int32)
        idx_ref[:, l] = idx
        onehot = (iota_k == idx[:, None]).astype(jnp.bfloat16)
        zq_ref[:, l * D : (l + 1) * D] = jnp.dot(
            onehot, cb_ref[...], preferred_element_type=jnp.float32
        )


def _decoder_kernel(zq_ref, w3_ref, b3_ref, w4_ref, b4_ref, out_ref):
    h = jnp.maximum(
        jnp.dot(zq_ref[...], w3_ref[...], preferred_element_type=jnp.float32)
        + b3_ref[...],
        0.0,
    )
    out_ref[...] = (
        jnp.dot(h, w4_ref[...], preferred_element_type=jnp.float32) + b4_ref[...]
    )


def _full(shape):
    return pl.BlockSpec(shape, lambda i: (0,) * len(shape))


def _rows(bt, cols):
    return pl.BlockSpec((bt, cols), lambda i: (i, 0))


@jax.jit
def kernel(x, W1, b1, W2, b2, codebook, W3, b3, W4, b4):
    b1r = b1.reshape(1, HIDDEN_DIM)
    b2r = b2.reshape(1, L * D)
    b3r = b3.reshape(1, HIDDEN_DIM)
    b4r = b4.reshape(1, INPUT_DIM)

    z = pl.pallas_call(
        _encoder_kernel,
        grid=(B // ENC_BT,),
        in_specs=[
            _rows(ENC_BT, INPUT_DIM),
            _full((INPUT_DIM, HIDDEN_DIM)),
            _full((1, HIDDEN_DIM)),
            _full((HIDDEN_DIM, L * D)),
            _full((1, L * D)),
        ],
        out_specs=_rows(ENC_BT, L * D),
        out_shape=jax.ShapeDtypeStruct((B, L * D), jnp.float32),
        compiler_params=pltpu.CompilerParams(
            dimension_semantics=("parallel",),
        ),
    )(x, W1, b1r, W2, b2r)

    z3 = z.reshape(B, L, D)
    c_sq = (codebook**2).sum(axis=1)
    zsq = jnp.concatenate(
        [(z3[:, i, :] ** 2).sum(axis=-1, keepdims=True) for i in range(L)], axis=1
    )
    cbt_bf16 = codebook.T.astype(jnp.bfloat16)
    cb_bf16 = codebook.astype(jnp.bfloat16)

    indices, z_q = pl.pallas_call(
        _vq_kernel,
        grid=(B // VQ_BT,),
        in_specs=[
            _rows(VQ_BT, L * D),
            _rows(VQ_BT, L),
            _full((1, K)),
            _full((D, K)),
            _full((K, D)),
        ],
        out_specs=[
            _rows(VQ_BT, L),
            _rows(VQ_BT, L * D),
        ],
        out_shape=[
            jax.ShapeDtypeStruct((B, L), jnp.int32),
            jax.ShapeDtypeStruct((B, L * D), jnp.float32),
        ],
        compiler_params=pltpu.CompilerParams(
            dimension_semantics=("parallel",),
        ),
    )(z, zsq, c_sq.reshape(1, K), cbt_bf16, cb_bf16)

    x_recon = pl.pallas_call(
        _decoder_kernel,
        grid=(B // DEC_BT,),
        in_specs=[
            _rows(DEC_BT, L * D),
            _full((L * D, HIDDEN_DIM)),
            _full((1, HIDDEN_DIM)),
            _full((HIDDEN_DIM, INPUT_DIM)),
            _full((1, INPUT_DIM)),
        ],
        out_specs=_rows(DEC_BT, INPUT_DIM),
        out_shape=jax.ShapeDtypeStruct((B, INPUT_DIM), jnp.float32),
        compiler_params=pltpu.CompilerParams(
            dimension_semantics=("parallel",),
        ),
    )(z_q, W3, b3r, W4, b4r)

    return (x_recon, z3, z_q, indices)
